# baseline (speedup 1.0000x reference)
import functools

import jax
import jax.numpy as jnp
from jax import lax
from jax.experimental import pallas as pl
from jax.experimental.pallas import tpu as pltpu
from jax.experimental.pallas import tpu_sc as plsc

_CHUNK = 128
_NUM_WORKERS = 32
_NUM_CORES = 2


@functools.partial(jax.jit, static_argnums=(2,))
def _lookup(flat_ids, table, n):
    per_w = n // _NUM_WORKERS
    chunks = per_w // _CHUNK
    mesh = plsc.VectorSubcoreMesh(core_axis_name="c", subcore_axis_name="s")

    @functools.partial(
        pl.kernel,
        mesh=mesh,
        out_type=jax.ShapeDtypeStruct((n,), jnp.float32),
        scratch_types=[
            pltpu.VMEM((per_w,), jnp.int32),
            pltpu.VMEM((per_w,), jnp.float32),
            pltpu.SemaphoreType.DMA,
        ],
    )
    def k(ids_hbm, tbl_hbm, out_hbm, ids_v, out_v, sem):
        wid = lax.axis_index("s") * _NUM_CORES + lax.axis_index("c")
        base = wid * per_w
        pltpu.sync_copy(ids_hbm.at[pl.ds(base, per_w)], ids_v)
        copies = [
            pltpu.async_copy(
                tbl_hbm.at[ids_v.at[pl.ds(j * _CHUNK, _CHUNK)]],
                out_v.at[pl.ds(j * _CHUNK, _CHUNK)],
                sem,
            )
            for j in range(chunks)
        ]
        for c in copies:
            c.wait()
        pltpu.sync_copy(out_v, out_hbm.at[pl.ds(base, per_w)])

    return k(flat_ids, table)


def kernel(entities, ent2cluster_key, ent2cluster_value):
    del ent2cluster_key
    b, l = entities.shape
    n = b * l
    flat = entities.reshape(n).astype(jnp.int32)
    out = _lookup(flat, ent2cluster_value.astype(jnp.float32), n)
    return out.reshape(b, l)


# speedup vs baseline: 1.0272x; 1.0272x over previous
import functools

import jax
import jax.numpy as jnp
from jax import lax
from jax.experimental import pallas as pl
from jax.experimental.pallas import tpu as pltpu
from jax.experimental.pallas import tpu_sc as plsc

_CHUNK = 128
_NUM_CORES = 1
_NUM_WORKERS = 16 * _NUM_CORES


@functools.partial(jax.jit, static_argnums=(2,))
def _lookup(flat_ids, table, n):
    per_w = n // _NUM_WORKERS
    chunks = per_w // _CHUNK
    mesh = plsc.VectorSubcoreMesh(
        core_axis_name="c", subcore_axis_name="s", num_cores=_NUM_CORES
    )

    @functools.partial(
        pl.kernel,
        mesh=mesh,
        out_type=jax.ShapeDtypeStruct((n,), jnp.float32),
        scratch_types=[
            pltpu.VMEM((per_w,), jnp.int32),
            pltpu.VMEM((per_w,), jnp.float32),
            pltpu.SemaphoreType.DMA,
        ],
    )
    def k(ids_hbm, tbl_hbm, out_hbm, ids_v, out_v, sem):
        wid = lax.axis_index("s") * _NUM_CORES + lax.axis_index("c")
        base = wid * per_w
        pltpu.sync_copy(ids_hbm.at[pl.ds(base, per_w)], ids_v)
        copies = [
            pltpu.async_copy(
                tbl_hbm.at[ids_v.at[pl.ds(j * _CHUNK, _CHUNK)]],
                out_v.at[pl.ds(j * _CHUNK, _CHUNK)],
                sem,
            )
            for j in range(chunks)
        ]
        for c in copies:
            c.wait()
        pltpu.sync_copy(out_v, out_hbm.at[pl.ds(base, per_w)])

    return k(flat_ids, table)


def kernel(entities, ent2cluster_key, ent2cluster_value):
    del ent2cluster_key
    b, l = entities.shape
    n = b * l
    flat = entities.reshape(n).astype(jnp.int32)
    out = _lookup(flat, ent2cluster_value.astype(jnp.float32), n)
    return out.reshape(b, l)


# speedup vs baseline: 1.5283x; 1.4879x over previous
import functools

import jax
import jax.numpy as jnp
from jax import lax
from jax.experimental import pallas as pl
from jax.experimental.pallas import tpu as pltpu
from jax.experimental.pallas import tpu_sc as plsc

_CHUNK = 128
_NUM_CORES = 2
_NUM_WORKERS = 16 * _NUM_CORES
_TBL_PAD = 1024


@functools.partial(jax.jit, static_argnums=(2,))
def _lookup(flat_ids, table, n):
    per_w = n // _NUM_WORKERS
    chunks = per_w // _CHUNK
    mesh = plsc.VectorSubcoreMesh(
        core_axis_name="c", subcore_axis_name="s", num_cores=_NUM_CORES
    )

    @functools.partial(
        pl.kernel,
        mesh=mesh,
        out_type=jax.ShapeDtypeStruct((n,), jnp.float32),
        scratch_types=[
            pltpu.VMEM((per_w,), jnp.int32),
            pltpu.VMEM((per_w,), jnp.float32),
            pltpu.VMEM_SHARED((_TBL_PAD,), jnp.float32),
            pltpu.SemaphoreType.DMA,
        ],
    )
    def k(ids_hbm, tbl_hbm, out_hbm, ids_v, out_v, tbl_s, sem):
        sid = lax.axis_index("s")
        wid = sid * _NUM_CORES + lax.axis_index("c")
        base = wid * per_w
        ids_cp = pltpu.async_copy(ids_hbm.at[pl.ds(base, per_w)], ids_v, sem)

        @pl.when(sid == 0)
        def _():
            pltpu.sync_copy(tbl_hbm, tbl_s)

        plsc.subcore_barrier()
        ids_cp.wait()
        copies = [
            pltpu.async_copy(
                tbl_s.at[ids_v.at[pl.ds(j * _CHUNK, _CHUNK)]],
                out_v.at[pl.ds(j * _CHUNK, _CHUNK)],
                sem,
            )
            for j in range(chunks)
        ]
        for c in copies:
            c.wait()
        pltpu.sync_copy(out_v, out_hbm.at[pl.ds(base, per_w)])

    return k(flat_ids, table)


def kernel(entities, ent2cluster_key, ent2cluster_value):
    del ent2cluster_key
    b, l = entities.shape
    n = b * l
    flat = entities.reshape(n).astype(jnp.int32)
    table = jnp.zeros((_TBL_PAD,), jnp.float32).at[: ent2cluster_value.shape[0]].set(
        ent2cluster_value.astype(jnp.float32)
    )
    out = _lookup(flat, table, n)
    return out.reshape(b, l)


# speedup vs baseline: 1.6260x; 1.0639x over previous
import functools

import jax
import jax.numpy as jnp
from jax import lax
from jax.experimental import pallas as pl
from jax.experimental.pallas import tpu as pltpu
from jax.experimental.pallas import tpu_sc as plsc

_CHUNK = 128
_NUM_CORES = 1
_NUM_WORKERS = 16 * _NUM_CORES
_TBL_PAD = 1024


@functools.partial(jax.jit, static_argnums=(2,))
def _lookup(flat_ids, table, n):
    per_w = n // _NUM_WORKERS
    chunks = per_w // _CHUNK
    mesh = plsc.VectorSubcoreMesh(
        core_axis_name="c", subcore_axis_name="s", num_cores=_NUM_CORES
    )

    @functools.partial(
        pl.kernel,
        mesh=mesh,
        out_type=jax.ShapeDtypeStruct((n,), jnp.float32),
        scratch_types=[
            pltpu.VMEM((per_w,), jnp.int32),
            pltpu.VMEM((per_w,), jnp.float32),
            pltpu.VMEM_SHARED((_TBL_PAD,), jnp.float32),
            pltpu.SemaphoreType.DMA,
        ],
    )
    def k(ids_hbm, tbl_hbm, out_hbm, ids_v, out_v, tbl_s, sem):
        sid = lax.axis_index("s")
        wid = sid * _NUM_CORES + lax.axis_index("c")
        base = wid * per_w
        ids_cp = pltpu.async_copy(ids_hbm.at[pl.ds(base, per_w)], ids_v, sem)

        @pl.when(sid == 0)
        def _():
            pltpu.sync_copy(tbl_hbm, tbl_s)

        plsc.subcore_barrier()
        ids_cp.wait()
        copies = [
            pltpu.async_copy(
                tbl_s.at[ids_v.at[pl.ds(j * _CHUNK, _CHUNK)]],
                out_v.at[pl.ds(j * _CHUNK, _CHUNK)],
                sem,
            )
            for j in range(chunks)
        ]
        for c in copies:
            c.wait()
        pltpu.sync_copy(out_v, out_hbm.at[pl.ds(base, per_w)])

    return k(flat_ids, table)


def kernel(entities, ent2cluster_key, ent2cluster_value):
    del ent2cluster_key
    b, l = entities.shape
    n = b * l
    flat = entities.reshape(n).astype(jnp.int32)
    table = jnp.zeros((_TBL_PAD,), jnp.float32).at[: ent2cluster_value.shape[0]].set(
        ent2cluster_value.astype(jnp.float32)
    )
    out = _lookup(flat, table, n)
    return out.reshape(b, l)
